# baseline (speedup 1.0000x reference)
import functools

import jax
import jax.numpy as jnp
from jax import lax
from jax.experimental import pallas as pl
from jax.experimental.pallas import tpu as pltpu
from jax.experimental.pallas import tpu_sc as plsc

NC = 2
NS = 16
NW = NC * NS
CH = 128
DW = 16


def _agg_sc(hs, srcp, dstp, n, d, cpt, agg_rows):
    zcpt = agg_rows // (NS * CH)
    ha = cpt // 2
    mesh = plsc.VectorSubcoreMesh(core_axis_name="c", subcore_axis_name="s")

    @functools.partial(
        pl.kernel,
        out_type=jax.ShapeDtypeStruct((NC, agg_rows, d), jnp.float32),
        mesh=mesh,
        scratch_types=[
            pltpu.VMEM((cpt, CH), jnp.int32),
            pltpu.VMEM((ha, CH), jnp.int32),
            pltpu.VMEM((CH, d), jnp.float32),
            pltpu.VMEM((CH, d), jnp.float32),
            pltpu.VMEM_SHARED((agg_rows, d), jnp.float32),
            pltpu.SemaphoreType.DMA,
            pltpu.SemaphoreType.DMA,
        ],
    )
    def k(hs_hbm, src_hbm, dst_hbm, out_hbm, src_v, dst_v, rows_a, rows_b,
          agg_sh, sem_a, sem_b):
        c = lax.axis_index("c")
        s = lax.axis_index("s")
        wid = s * NC + c

        def zrow(i, carry):
            for kk in range(d // 16):
                rows_a[i, pl.ds(kk * 16, 16)] = jnp.zeros((16,), jnp.float32)
            return carry

        lax.fori_loop(0, CH, zrow, 0)
        for kk in range(zcpt):
            pltpu.sync_copy(rows_a, agg_sh.at[pl.ds(s * zcpt * CH + kk * CH, CH)])

        pltpu.sync_copy(src_hbm.at[wid], src_v)
        pltpu.sync_copy(dst_hbm.at[wid, pl.ds(0, ha)], dst_v)
        plsc.subcore_barrier()

        last = cpt - 1

        def gather(j, buf, sm):
            pltpu.async_copy(hs_hbm.at[src_v.at[jnp.minimum(j, last)]], buf, sm)

        def wait_a():
            pltpu.make_async_copy(hs_hbm.at[src_v.at[0]], rows_a, sem_a).wait()

        def wait_b():
            pltpu.make_async_copy(hs_hbm.at[src_v.at[0]], rows_b, sem_b).wait()

        def pair(j, doff):
            gather(j + 1, rows_b, sem_b)
            wait_a()
            pltpu.sync_copy(rows_a, agg_sh.at[dst_v.at[j - doff]], add=True)
            gather(j + 2, rows_a, sem_a)
            wait_b()
            pltpu.sync_copy(rows_b, agg_sh.at[dst_v.at[j + 1 - doff]], add=True)

        gather(0, rows_a, sem_a)

        def body_a(j2, carry):
            pair(2 * j2, 0)
            return carry

        lax.fori_loop(0, ha // 2, body_a, 0)

        pltpu.sync_copy(dst_hbm.at[wid, pl.ds(ha, ha)], dst_v)

        def body_b(j2, carry):
            pair(ha + 2 * j2, ha)
            return carry

        lax.fori_loop(0, ha // 2, body_b, 0)
        wait_a()
        plsc.subcore_barrier()

        for kk in range(zcpt):
            r0 = s * zcpt * CH + kk * CH
            pltpu.sync_copy(agg_sh.at[pl.ds(r0, CH)], rows_a)
            pltpu.sync_copy(rows_a, out_hbm.at[c, pl.ds(r0, CH)])

    return k(hs, srcp, dstp)


def _deg_sc(dstp, n, cpt, agg_rows):
    zcpt = agg_rows // (NS * CH)
    degw = 128
    mesh = plsc.VectorSubcoreMesh(core_axis_name="c", subcore_axis_name="s")

    @functools.partial(
        pl.kernel,
        out_type=jax.ShapeDtypeStruct((NC, agg_rows, degw), jnp.float32),
        mesh=mesh,
        scratch_types=[
            pltpu.VMEM((cpt, CH), jnp.int32),
            pltpu.VMEM((CH, degw), jnp.float32),
            pltpu.VMEM_SHARED((agg_rows, degw), jnp.float32),
            pltpu.SemaphoreType.DMA,
        ],
    )
    def k(dst_hbm, out_hbm, dst_v, ones_v, deg_sh, sem):
        c = lax.axis_index("c")
        s = lax.axis_index("s")
        wid = s * NC + c

        def fill(val):
            def frow(i, carry):
                for kk in range(degw // 16):
                    ones_v[i, pl.ds(kk * 16, 16)] = jnp.full((16,), val, jnp.float32)
                return carry

            lax.fori_loop(0, CH, frow, 0)

        fill(0.0)
        for kk in range(zcpt):
            pltpu.sync_copy(ones_v, deg_sh.at[pl.ds(s * zcpt * CH + kk * CH, CH)])
        fill(1.0)
        pltpu.sync_copy(dst_hbm.at[wid], dst_v)
        plsc.subcore_barrier()

        def body(j, carry):
            pltpu.sync_copy(ones_v, deg_sh.at[dst_v.at[j]], add=True)
            return carry

        lax.fori_loop(0, cpt, body, 0)
        plsc.subcore_barrier()

        for kk in range(zcpt):
            r0 = s * zcpt * CH + kk * CH
            pltpu.sync_copy(deg_sh.at[pl.ds(r0, CH)], ones_v)
            pltpu.sync_copy(ones_v, out_hbm.at[c, pl.ds(r0, CH)])

    return k(dstp)


def _tc_first(x, w, degp, n, r):
    g = n // r
    din, dout = w.shape

    def body(x_ref, w_ref, deg_ref, hs_ref, dinv_ref):
        deg = deg_ref[0] + deg_ref[1] + 1.0
        dinv = lax.rsqrt(deg)
        h = jnp.dot(x_ref[...], w_ref[...], preferred_element_type=jnp.float32)
        hs_ref[...] = h * dinv[:, 0:1]
        dinv_ref[...] = dinv[:, 0:DW]

    return pl.pallas_call(
        body,
        grid=(g,),
        in_specs=[
            pl.BlockSpec((r, din), lambda i: (i, 0)),
            pl.BlockSpec((din, dout), lambda i: (0, 0)),
            pl.BlockSpec((NC, r, 128), lambda i: (0, i, 0)),
        ],
        out_specs=[
            pl.BlockSpec((r, dout), lambda i: (i, 0)),
            pl.BlockSpec((r, DW), lambda i: (i, 0)),
        ],
        out_shape=[
            jax.ShapeDtypeStruct((n, dout), jnp.float32),
            jax.ShapeDtypeStruct((n, DW), jnp.float32),
        ],
    )(x, w, degp)


def _tc_mid(parts, hs, dinv, b, w, n, r):
    g = n // r
    din, dout = w.shape

    def body(p_ref, hs_ref, dinv_ref, b_ref, w_ref, o_ref):
        s = p_ref[0] + p_ref[1] + hs_ref[...]
        z = s * dinv_ref[:, 0:1] + b_ref[...]
        z = jnp.maximum(z, 0.0)
        h = jnp.dot(z, w_ref[...], preferred_element_type=jnp.float32)
        o_ref[...] = h * dinv_ref[:, 0:1]

    return pl.pallas_call(
        body,
        grid=(g,),
        in_specs=[
            pl.BlockSpec((NC, r, din), lambda i: (0, i, 0)),
            pl.BlockSpec((r, din), lambda i: (i, 0)),
            pl.BlockSpec((r, DW), lambda i: (i, 0)),
            pl.BlockSpec((1, din), lambda i: (0, 0)),
            pl.BlockSpec((din, dout), lambda i: (0, 0)),
        ],
        out_specs=pl.BlockSpec((r, dout), lambda i: (i, 0)),
        out_shape=jax.ShapeDtypeStruct((n, dout), jnp.float32),
    )(parts, hs, dinv, b, w)


def _tc_pre(parts, hs, dinv, b, n, r):
    g = n // r
    d = hs.shape[1]

    def body(p_ref, hs_ref, dinv_ref, b_ref, o_ref):
        s = p_ref[0] + p_ref[1] + hs_ref[...]
        z = s * dinv_ref[:, 0:1] + b_ref[...]
        z = jnp.maximum(z, 0.0)
        o_ref[...] = z * dinv_ref[:, 0:1]

    return pl.pallas_call(
        body,
        grid=(g,),
        in_specs=[
            pl.BlockSpec((NC, r, d), lambda i: (0, i, 0)),
            pl.BlockSpec((r, d), lambda i: (i, 0)),
            pl.BlockSpec((r, DW), lambda i: (i, 0)),
            pl.BlockSpec((1, d), lambda i: (0, 0)),
        ],
        out_specs=pl.BlockSpec((r, d), lambda i: (i, 0)),
        out_shape=jax.ShapeDtypeStruct((n, d), jnp.float32),
    )(parts, hs, dinv, b)


def _tc_last(parts, zs, dinv, b, w, n, r):
    g = n // r
    din, dout = w.shape

    def body(p_ref, zs_ref, dinv_ref, b_ref, w_ref, o_ref):
        s = p_ref[0] + p_ref[1] + zs_ref[...]
        t = jnp.dot(s, w_ref[...], preferred_element_type=jnp.float32)
        z = t * dinv_ref[:, 0:1] + b_ref[...]
        m = jnp.max(z, axis=1, keepdims=True)
        lse = jnp.log(jnp.sum(jnp.exp(z - m), axis=1, keepdims=True)) + m
        o_ref[...] = z - lse

    return pl.pallas_call(
        body,
        grid=(g,),
        in_specs=[
            pl.BlockSpec((NC, r, din), lambda i: (0, i, 0)),
            pl.BlockSpec((r, din), lambda i: (i, 0)),
            pl.BlockSpec((r, DW), lambda i: (i, 0)),
            pl.BlockSpec((1, dout), lambda i: (0, 0)),
            pl.BlockSpec((din, dout), lambda i: (0, 0)),
        ],
        out_specs=pl.BlockSpec((r, dout), lambda i: (i, 0)),
        out_shape=jax.ShapeDtypeStruct((n, dout), jnp.float32),
    )(parts, zs, dinv, b, w)


def kernel(x, edge_index, W1, b1, Wh, bh, W2, b2):
    n, d_in = x.shape
    e = edge_index.shape[1]
    d_h = W1.shape[1]
    d_out = W2.shape[1]

    cpt = 4 * (-(-e // (NW * CH * 4)))
    epad = NW * cpt * CH
    agg_rows = NS * CH * (-(-(n + 1) // (NS * CH)))
    r = 2000

    src = edge_index[0]
    dst = edge_index[1]
    pad = epad - e
    srcp = jnp.concatenate([src, jnp.zeros((pad,), jnp.int32)]).reshape(NW, cpt, CH)
    dstp = jnp.concatenate([dst, jnp.full((pad,), n, jnp.int32)]).reshape(NW, cpt, CH)

    degp = _deg_sc(dstp, n, cpt, agg_rows)
    hs1, dinv = _tc_first(x, W1, degp, n, r)
    s1 = _agg_sc(hs1, srcp, dstp, n, d_h, cpt, agg_rows)
    hs2 = _tc_mid(s1, hs1, dinv, b1.reshape(1, d_h), Wh, n, r)
    s2 = _agg_sc(hs2, srcp, dstp, n, d_h, cpt, agg_rows)
    zs2 = _tc_pre(s2, hs2, dinv, bh.reshape(1, d_h), n, r)
    s3 = _agg_sc(zs2, srcp, dstp, n, d_h, cpt, agg_rows)
    return _tc_last(s3, zs2, dinv, b2.reshape(1, d_out), W2, n, r)


# speedup vs baseline: 1.4107x; 1.4107x over previous
import functools

import jax
import jax.numpy as jnp
from jax import lax
from jax.experimental import pallas as pl
from jax.experimental.pallas import tpu as pltpu
from jax.experimental.pallas import tpu_sc as plsc

NC = 2
NS = 16
NW = NC * NS
CH = 128
DW = 16


def _agg_sc(hs, srcp, dstp, n, d, cpt, agg_rows):
    zcpt = agg_rows // (NS * CH)
    mesh = plsc.VectorSubcoreMesh(core_axis_name="c", subcore_axis_name="s")

    @functools.partial(
        pl.kernel,
        out_type=jax.ShapeDtypeStruct((NC, agg_rows, d), jnp.float32),
        mesh=mesh,
        scratch_types=[
            pltpu.VMEM((cpt, CH), jnp.int32),
            pltpu.VMEM((cpt, CH), jnp.int32),
            pltpu.VMEM((CH, d), jnp.float32),
            pltpu.VMEM_SHARED((agg_rows, d), jnp.float32),
            pltpu.SemaphoreType.DMA,
        ],
    )
    def k(hs_hbm, src_hbm, dst_hbm, out_hbm, src_v, dst_v, rows_v,
          agg_sh, sem):
        c = lax.axis_index("c")
        s = lax.axis_index("s")
        wid = s * NC + c

        def zrow(i, carry):
            for kk in range(d // 16):
                rows_v[i, pl.ds(kk * 16, 16)] = jnp.zeros((16,), jnp.float32)
            return carry

        lax.fori_loop(0, CH, zrow, 0)
        for kk in range(zcpt):
            pltpu.sync_copy(rows_v, agg_sh.at[pl.ds(s * zcpt * CH + kk * CH, CH)])

        pltpu.sync_copy(src_hbm.at[wid], src_v)
        pltpu.sync_copy(dst_hbm.at[wid], dst_v)
        plsc.subcore_barrier()

        def body(j, carry):
            pltpu.async_copy(hs_hbm.at[src_v.at[j]], rows_v, sem).wait()
            pltpu.sync_copy(rows_v, agg_sh.at[dst_v.at[j]], add=True)
            return carry

        lax.fori_loop(0, cpt, body, 0)
        plsc.subcore_barrier()

        for kk in range(zcpt):
            r0 = s * zcpt * CH + kk * CH
            pltpu.sync_copy(agg_sh.at[pl.ds(r0, CH)], rows_v)
            pltpu.sync_copy(rows_v, out_hbm.at[c, pl.ds(r0, CH)])

    return k(hs, srcp, dstp)


def _deg_sc(dstp, n, cpt, agg_rows):
    zcpt = agg_rows // (NS * CH)
    degw = 128
    mesh = plsc.VectorSubcoreMesh(core_axis_name="c", subcore_axis_name="s")

    @functools.partial(
        pl.kernel,
        out_type=jax.ShapeDtypeStruct((NC, agg_rows, degw), jnp.float32),
        mesh=mesh,
        scratch_types=[
            pltpu.VMEM((cpt, CH), jnp.int32),
            pltpu.VMEM((CH, degw), jnp.float32),
            pltpu.VMEM_SHARED((agg_rows, degw), jnp.float32),
            pltpu.SemaphoreType.DMA,
        ],
    )
    def k(dst_hbm, out_hbm, dst_v, ones_v, deg_sh, sem):
        c = lax.axis_index("c")
        s = lax.axis_index("s")
        wid = s * NC + c

        def fill(val):
            def frow(i, carry):
                for kk in range(degw // 16):
                    ones_v[i, pl.ds(kk * 16, 16)] = jnp.full((16,), val, jnp.float32)
                return carry

            lax.fori_loop(0, CH, frow, 0)

        fill(0.0)
        for kk in range(zcpt):
            pltpu.sync_copy(ones_v, deg_sh.at[pl.ds(s * zcpt * CH + kk * CH, CH)])
        fill(1.0)
        pltpu.sync_copy(dst_hbm.at[wid], dst_v)
        plsc.subcore_barrier()

        def body(j, carry):
            pltpu.sync_copy(ones_v, deg_sh.at[dst_v.at[j]], add=True)
            return carry

        lax.fori_loop(0, cpt, body, 0)
        plsc.subcore_barrier()

        for kk in range(zcpt):
            r0 = s * zcpt * CH + kk * CH
            pltpu.sync_copy(deg_sh.at[pl.ds(r0, CH)], ones_v)
            pltpu.sync_copy(ones_v, out_hbm.at[c, pl.ds(r0, CH)])

    return k(dstp)


def _tc_first(x, w, degp, n, r):
    g = n // r
    din, dout = w.shape

    def body(x_ref, w_ref, deg_ref, hs_ref, dinv_ref):
        deg = deg_ref[0] + deg_ref[1] + 1.0
        dinv = lax.rsqrt(deg)
        h = jnp.dot(x_ref[...], w_ref[...], preferred_element_type=jnp.float32)
        hs_ref[...] = h * dinv[:, 0:1]
        dinv_ref[...] = dinv[:, 0:DW]

    return pl.pallas_call(
        body,
        grid=(g,),
        in_specs=[
            pl.BlockSpec((r, din), lambda i: (i, 0)),
            pl.BlockSpec((din, dout), lambda i: (0, 0)),
            pl.BlockSpec((NC, r, 128), lambda i: (0, i, 0)),
        ],
        out_specs=[
            pl.BlockSpec((r, dout), lambda i: (i, 0)),
            pl.BlockSpec((r, DW), lambda i: (i, 0)),
        ],
        out_shape=[
            jax.ShapeDtypeStruct((n, dout), jnp.float32),
            jax.ShapeDtypeStruct((n, DW), jnp.float32),
        ],
    )(x, w, degp)


def _tc_mid(parts, hs, dinv, b, w, n, r):
    g = n // r
    din, dout = w.shape

    def body(p_ref, hs_ref, dinv_ref, b_ref, w_ref, o_ref):
        s = p_ref[0] + p_ref[1] + hs_ref[...]
        z = s * dinv_ref[:, 0:1] + b_ref[...]
        z = jnp.maximum(z, 0.0)
        h = jnp.dot(z, w_ref[...], preferred_element_type=jnp.float32)
        o_ref[...] = h * dinv_ref[:, 0:1]

    return pl.pallas_call(
        body,
        grid=(g,),
        in_specs=[
            pl.BlockSpec((NC, r, din), lambda i: (0, i, 0)),
            pl.BlockSpec((r, din), lambda i: (i, 0)),
            pl.BlockSpec((r, DW), lambda i: (i, 0)),
            pl.BlockSpec((1, din), lambda i: (0, 0)),
            pl.BlockSpec((din, dout), lambda i: (0, 0)),
        ],
        out_specs=pl.BlockSpec((r, dout), lambda i: (i, 0)),
        out_shape=jax.ShapeDtypeStruct((n, dout), jnp.float32),
    )(parts, hs, dinv, b, w)


def _tc_pre(parts, hs, dinv, b, n, r):
    g = n // r
    d = hs.shape[1]

    def body(p_ref, hs_ref, dinv_ref, b_ref, o_ref):
        s = p_ref[0] + p_ref[1] + hs_ref[...]
        z = s * dinv_ref[:, 0:1] + b_ref[...]
        z = jnp.maximum(z, 0.0)
        o_ref[...] = z * dinv_ref[:, 0:1]

    return pl.pallas_call(
        body,
        grid=(g,),
        in_specs=[
            pl.BlockSpec((NC, r, d), lambda i: (0, i, 0)),
            pl.BlockSpec((r, d), lambda i: (i, 0)),
            pl.BlockSpec((r, DW), lambda i: (i, 0)),
            pl.BlockSpec((1, d), lambda i: (0, 0)),
        ],
        out_specs=pl.BlockSpec((r, d), lambda i: (i, 0)),
        out_shape=jax.ShapeDtypeStruct((n, d), jnp.float32),
    )(parts, hs, dinv, b)


def _tc_last(parts, zs, dinv, b, w, n, r):
    g = n // r
    din, dout = w.shape

    def body(p_ref, zs_ref, dinv_ref, b_ref, w_ref, o_ref):
        s = p_ref[0] + p_ref[1] + zs_ref[...]
        t = jnp.dot(s, w_ref[...], preferred_element_type=jnp.float32)
        z = t * dinv_ref[:, 0:1] + b_ref[...]
        m = jnp.max(z, axis=1, keepdims=True)
        lse = jnp.log(jnp.sum(jnp.exp(z - m), axis=1, keepdims=True)) + m
        o_ref[...] = z - lse

    return pl.pallas_call(
        body,
        grid=(g,),
        in_specs=[
            pl.BlockSpec((NC, r, din), lambda i: (0, i, 0)),
            pl.BlockSpec((r, din), lambda i: (i, 0)),
            pl.BlockSpec((r, DW), lambda i: (i, 0)),
            pl.BlockSpec((1, dout), lambda i: (0, 0)),
            pl.BlockSpec((din, dout), lambda i: (0, 0)),
        ],
        out_specs=pl.BlockSpec((r, dout), lambda i: (i, 0)),
        out_shape=jax.ShapeDtypeStruct((n, dout), jnp.float32),
    )(parts, zs, dinv, b, w)


def kernel(x, edge_index, W1, b1, Wh, bh, W2, b2):
    n, d_in = x.shape
    e = edge_index.shape[1]
    d_h = W1.shape[1]
    d_out = W2.shape[1]

    cpt = -(-e // (NW * CH))
    epad = NW * cpt * CH
    agg_rows = NS * CH * (-(-(n + 1) // (NS * CH)))
    r = 2000

    src = edge_index[0]
    dst = edge_index[1]
    pad = epad - e
    srcp = jnp.concatenate([src, jnp.zeros((pad,), jnp.int32)]).reshape(NW, cpt, CH)
    dstp = jnp.concatenate([dst, jnp.full((pad,), n, jnp.int32)]).reshape(NW, cpt, CH)

    degp = _deg_sc(dstp, n, cpt, agg_rows)
    hs1, dinv = _tc_first(x, W1, degp, n, r)
    s1 = _agg_sc(hs1, srcp, dstp, n, d_h, cpt, agg_rows)
    hs2 = _tc_mid(s1, hs1, dinv, b1.reshape(1, d_h), Wh, n, r)
    s2 = _agg_sc(hs2, srcp, dstp, n, d_h, cpt, agg_rows)
    zs2 = _tc_pre(s2, hs2, dinv, bh.reshape(1, d_h), n, r)
    s3 = _agg_sc(zs2, srcp, dstp, n, d_h, cpt, agg_rows)
    return _tc_last(s3, zs2, dinv, b2.reshape(1, d_out), W2, n, r)
